# baseline (speedup 1.0000x reference)
import functools

import jax
import jax.numpy as jnp
from jax import lax
from jax.experimental import pallas as pl
from jax.experimental.pallas import tpu as pltpu
from jax.experimental.pallas import tpu_sc as plsc

_LANES = 512
_GRID = 4
_STREAMS = 2
_SC_IMGS = 4
_NW = 32
_UNROLL = 8

_IMG = 512 * 512
_TOTAL_ROWS = 16 * _IMG // _LANES
_TC_ROWS = (16 - _SC_IMGS) * _IMG // _LANES
_SC_ROWS_PER_W = (_TOTAL_ROWS - _TC_ROWS) // _NW


def _tc_l1_sum_kernel(*refs):
    *in_refs, o_ref, acc_ref = refs
    i = pl.program_id(0)

    @pl.when(i == 0)
    def _init():
        acc_ref[...] = jnp.zeros_like(acc_ref)

    n = len(in_refs) // 2
    part = jnp.zeros_like(acc_ref[...])
    for s in range(n):
        part += jnp.sum(jnp.abs(in_refs[s][...] - in_refs[n + s][...]),
                        axis=0, keepdims=True)
    acc_ref[...] += part

    @pl.when(i == pl.num_programs(0) - 1)
    def _finish():
        o_ref[0, 0] = jnp.sum(acc_ref[...])


def _sc_l1_sum_kernel(p_hbm, g_hbm, out_hbm, p_v, g_v, acc_v):
    wid = lax.axis_index("s") * 2 + lax.axis_index("c")
    base = _TC_ROWS + wid * _SC_ROWS_PER_W
    pltpu.sync_copy(p_hbm.at[pl.ds(base, _SC_ROWS_PER_W)], p_v)
    pltpu.sync_copy(g_hbm.at[pl.ds(base, _SC_ROWS_PER_W)], g_v)

    n_slices = _LANES // 16

    def inner(r, accs):
        out = list(accs)
        for u in range(n_slices):
            sl = pl.ds(u * 16, 16)
            out[u % _UNROLL] = (out[u % _UNROLL]
                                + jnp.abs(p_v[r, sl] - g_v[r, sl]))
        return tuple(out)

    zero = jnp.zeros((16,), jnp.float32)
    accs = lax.fori_loop(0, _SC_ROWS_PER_W, inner, (zero,) * _UNROLL)
    total = accs[0]
    for u in range(1, _UNROLL):
        total = total + accs[u]
    acc_v[...] = total
    pltpu.sync_copy(acc_v, out_hbm.at[pl.ds(wid * 16, 16)])


def kernel(pred_logits, gt_sdf):
    p = pred_logits.reshape(-1, _LANES)
    g = gt_sdf.reshape(-1, _LANES)

    blk = _TC_ROWS // (_STREAMS * _GRID)
    specs = [
        pl.BlockSpec((blk, _LANES), lambda i, s=s: (i + s * _GRID, 0))
        for s in range(_STREAMS)
    ]
    tc_total = pl.pallas_call(
        _tc_l1_sum_kernel,
        grid=(_GRID,),
        in_specs=specs + specs,
        out_specs=pl.BlockSpec(memory_space=pltpu.SMEM),
        out_shape=jax.ShapeDtypeStruct((1, 1), jnp.float32),
        scratch_shapes=[pltpu.VMEM((1, _LANES), jnp.float32)],
    )(*([p] * _STREAMS), *([g] * _STREAMS))

    sc_call = functools.partial(
        pl.kernel,
        mesh=plsc.VectorSubcoreMesh(core_axis_name="c", subcore_axis_name="s"),
        out_type=jax.ShapeDtypeStruct((_NW * 16,), jnp.float32),
        scratch_types=[
            pltpu.VMEM((_SC_ROWS_PER_W, _LANES), jnp.float32),
            pltpu.VMEM((_SC_ROWS_PER_W, _LANES), jnp.float32),
            pltpu.VMEM((16,), jnp.float32),
        ],
    )(_sc_l1_sum_kernel)
    sc_part = sc_call(p, g)

    return (tc_total[0, 0] + jnp.sum(sc_part)) * (1.0 / p.size)


# speedup vs baseline: 2.7776x; 2.7776x over previous
import jax
import jax.numpy as jnp
from jax.experimental import pallas as pl
from jax.experimental.pallas import tpu as pltpu

_LANES = 512
_GRID = 4
_STREAMS = 2


def _l1_mean_kernel(inv_n_ref, *refs):
    *in_refs, o_ref, acc_ref = refs
    i = pl.program_id(0)

    @pl.when(i == 0)
    def _init():
        acc_ref[...] = jnp.zeros_like(acc_ref)

    n = len(in_refs) // 2
    part = jnp.zeros_like(acc_ref[...])
    for s in range(n):
        part += jnp.sum(jnp.abs(in_refs[s][...] - in_refs[n + s][...]),
                        axis=0, keepdims=True)
    acc_ref[...] += part

    @pl.when(i == pl.num_programs(0) - 1)
    def _finish():
        o_ref[0, 0] = jnp.sum(acc_ref[...]) * inv_n_ref[0]


def kernel(pred_logits, gt_sdf):
    p = pred_logits.reshape(-1, _LANES)
    g = gt_sdf.reshape(-1, _LANES)
    rows = p.shape[0]
    blk = rows // (_STREAMS * _GRID)
    inv_n = jnp.full((1,), 1.0 / p.size, dtype=jnp.float32)
    specs = [
        pl.BlockSpec((blk, _LANES), lambda i, s=s: (i + s * _GRID, 0))
        for s in range(_STREAMS)
    ]
    total = pl.pallas_call(
        _l1_mean_kernel,
        grid=(_GRID,),
        in_specs=[pl.BlockSpec(memory_space=pltpu.SMEM)] + specs + specs,
        out_specs=pl.BlockSpec(memory_space=pltpu.SMEM),
        out_shape=jax.ShapeDtypeStruct((1, 1), jnp.float32),
        scratch_shapes=[pltpu.VMEM((1, _LANES), jnp.float32)],
    )(inv_n, *([p] * _STREAMS), *([g] * _STREAMS))
    return total[0, 0]
